# baseline (speedup 1.0000x reference)
import jax
import jax.numpy as jnp
from jax.experimental import pallas as pl
from jax.experimental.pallas import tpu as pltpu

Q = 1024
D = 64
BK = 8192
T = BK // 128
RQ = 128
INT_MAX = jnp.iinfo(jnp.int32).max


def _nn_kernel(qsq_ref, qm2_ref, k_ref, ksq_ref, out_ref, val_ref, idx_ref):
    i = pl.program_id(0)
    nb = pl.num_programs(0)

    @pl.when(i == 0)
    def _init():
        val_ref[...] = jnp.full((Q, 128), jnp.inf, dtype=jnp.float32)
        idx_ref[...] = jnp.zeros((Q, 128), dtype=jnp.int32)

    m2 = jax.lax.dot_general(
        qm2_ref[...].astype(jnp.bfloat16),
        k_ref[...].astype(jnp.bfloat16), (((1,), (0,)), ((), ())),
        preferred_element_type=jnp.float32)

    for r in range(Q // RQ):
        rs = slice(r * RQ, (r + 1) * RQ)
        qsq = qsq_ref[rs, :]
        val = val_ref[rs, :]
        idx = idx_ref[rs, :]
        for t in range(T):
            sl = slice(t * 128, (t + 1) * 128)
            v = (qsq + m2[rs, sl]) + ksq_ref[:, sl]
            tg = i * T + t
            mask = v < val
            val = jnp.where(mask, v, val)
            idx = jnp.where(mask, tg, idx)
        val_ref[rs, :] = val
        idx_ref[rs, :] = idx

    @pl.when(i == nb - 1)
    def _finish():
        valf = val_ref[...]
        idxf = idx_ref[...]
        gmin = jnp.min(valf, axis=1, keepdims=True)
        lane = jax.lax.broadcasted_iota(jnp.int32, (Q, 128), 1)
        full_idx = idxf * 128 + lane
        out_ref[...] = jnp.min(
            jnp.where(valf == gmin, full_idx, INT_MAX),
            axis=1, keepdims=True)


@jax.jit
def kernel(queries, keys):
    n_keys = keys.shape[0]
    n_blocks = pl.cdiv(n_keys, BK)
    k_pad = n_blocks * BK

    qsq = jnp.sum(queries * queries, axis=1, keepdims=True)
    qm2 = -2.0 * queries
    ksq = jnp.sum(keys * keys, axis=1)
    ksq = jnp.pad(ksq, (0, k_pad - n_keys),
                  constant_values=jnp.inf)[None, :]
    keys_t = keys.T

    out = pl.pallas_call(
        _nn_kernel,
        grid=(n_blocks,),
        in_specs=[
            pl.BlockSpec((Q, 1), lambda i: (0, 0)),
            pl.BlockSpec((Q, D), lambda i: (0, 0)),
            pl.BlockSpec((D, BK), lambda i: (0, i)),
            pl.BlockSpec((1, BK), lambda i: (0, i)),
        ],
        out_specs=pl.BlockSpec((Q, 1), lambda i: (0, 0)),
        out_shape=jax.ShapeDtypeStruct((Q, 1), jnp.int32),
        scratch_shapes=[pltpu.VMEM((Q, 128), jnp.float32),
                        pltpu.VMEM((Q, 128), jnp.int32)],
    )(qsq, qm2, keys_t, ksq)
    return out[:, 0]


# speedup vs baseline: 1.0487x; 1.0487x over previous
import jax
import jax.numpy as jnp
from jax.experimental import pallas as pl
from jax.experimental.pallas import tpu as pltpu

Q = 1024
D = 64
BK = 12544
T = BK // 128
RQ = 128
INT_MAX = jnp.iinfo(jnp.int32).max


def _nn_kernel(qsq_ref, qm2_ref, k_ref, ksq_ref, out_ref, val_ref, idx_ref):
    i = pl.program_id(0)
    nb = pl.num_programs(0)

    @pl.when(i == 0)
    def _init():
        val_ref[...] = jnp.full((Q, 128), jnp.inf, dtype=jnp.float32)
        idx_ref[...] = jnp.zeros((Q, 128), dtype=jnp.int32)

    m2 = jax.lax.dot_general(
        qm2_ref[...].astype(jnp.bfloat16),
        k_ref[...].astype(jnp.bfloat16), (((1,), (0,)), ((), ())),
        preferred_element_type=jnp.float32)

    for r in range(Q // RQ):
        rs = slice(r * RQ, (r + 1) * RQ)
        qsq = qsq_ref[rs, :]
        val = val_ref[rs, :]
        idx = idx_ref[rs, :]
        for t in range(T):
            sl = slice(t * 128, (t + 1) * 128)
            v = (qsq + m2[rs, sl]) + ksq_ref[:, sl]
            tg = i * T + t
            mask = v < val
            val = jnp.where(mask, v, val)
            idx = jnp.where(mask, tg, idx)
        val_ref[rs, :] = val
        idx_ref[rs, :] = idx

    @pl.when(i == nb - 1)
    def _finish():
        valf = val_ref[...]
        idxf = idx_ref[...]
        gmin = jnp.min(valf, axis=1, keepdims=True)
        lane = jax.lax.broadcasted_iota(jnp.int32, (Q, 128), 1)
        full_idx = idxf * 128 + lane
        out_ref[...] = jnp.min(
            jnp.where(valf == gmin, full_idx, INT_MAX),
            axis=1, keepdims=True)


@jax.jit
def kernel(queries, keys):
    n_keys = keys.shape[0]
    n_blocks = pl.cdiv(n_keys, BK)
    k_pad = n_blocks * BK

    qsq = jnp.sum(queries * queries, axis=1, keepdims=True)
    qm2 = -2.0 * queries
    ksq = jnp.sum(keys * keys, axis=1)
    ksq = jnp.pad(ksq, (0, k_pad - n_keys),
                  constant_values=jnp.inf)[None, :]
    keys_t = keys.T

    out = pl.pallas_call(
        _nn_kernel,
        grid=(n_blocks,),
        in_specs=[
            pl.BlockSpec((Q, 1), lambda i: (0, 0)),
            pl.BlockSpec((Q, D), lambda i: (0, 0)),
            pl.BlockSpec((D, BK), lambda i: (0, i)),
            pl.BlockSpec((1, BK), lambda i: (0, i)),
        ],
        out_specs=pl.BlockSpec((Q, 1), lambda i: (0, 0)),
        out_shape=jax.ShapeDtypeStruct((Q, 1), jnp.int32),
        scratch_shapes=[pltpu.VMEM((Q, 128), jnp.float32),
                        pltpu.VMEM((Q, 128), jnp.int32)],
    )(qsq, qm2, keys_t, ksq)
    return out[:, 0]


# speedup vs baseline: 1.0541x; 1.0051x over previous
import jax
import jax.numpy as jnp
from jax.experimental import pallas as pl
from jax.experimental.pallas import tpu as pltpu

Q = 1024
D = 64
BK = 14336
T = BK // 128
RQ = 128
INT_MAX = jnp.iinfo(jnp.int32).max


def _nn_kernel(qsq_ref, qm2_ref, k_ref, ksq_ref, out_ref, val_ref, idx_ref):
    i = pl.program_id(0)
    nb = pl.num_programs(0)

    @pl.when(i == 0)
    def _init():
        val_ref[...] = jnp.full((Q, 128), jnp.inf, dtype=jnp.float32)
        idx_ref[...] = jnp.zeros((Q, 128), dtype=jnp.int32)

    m2 = jax.lax.dot_general(
        qm2_ref[...].astype(jnp.bfloat16),
        k_ref[...].astype(jnp.bfloat16), (((1,), (0,)), ((), ())),
        preferred_element_type=jnp.float32)

    for r in range(Q // RQ):
        rs = slice(r * RQ, (r + 1) * RQ)
        qsq = qsq_ref[rs, :]
        val = val_ref[rs, :]
        idx = idx_ref[rs, :]
        for t in range(T):
            sl = slice(t * 128, (t + 1) * 128)
            v = (qsq + m2[rs, sl]) + ksq_ref[:, sl]
            tg = i * T + t
            mask = v < val
            val = jnp.where(mask, v, val)
            idx = jnp.where(mask, tg, idx)
        val_ref[rs, :] = val
        idx_ref[rs, :] = idx

    @pl.when(i == nb - 1)
    def _finish():
        valf = val_ref[...]
        idxf = idx_ref[...]
        gmin = jnp.min(valf, axis=1, keepdims=True)
        lane = jax.lax.broadcasted_iota(jnp.int32, (Q, 128), 1)
        full_idx = idxf * 128 + lane
        out_ref[...] = jnp.min(
            jnp.where(valf == gmin, full_idx, INT_MAX),
            axis=1, keepdims=True)


@jax.jit
def kernel(queries, keys):
    n_keys = keys.shape[0]
    n_blocks = pl.cdiv(n_keys, BK)
    k_pad = n_blocks * BK

    qsq = jnp.sum(queries * queries, axis=1, keepdims=True)
    qm2 = -2.0 * queries
    ksq = jnp.sum(keys * keys, axis=1)
    ksq = jnp.pad(ksq, (0, k_pad - n_keys),
                  constant_values=jnp.inf)[None, :]
    keys_t = keys.T

    out = pl.pallas_call(
        _nn_kernel,
        grid=(n_blocks,),
        in_specs=[
            pl.BlockSpec((Q, 1), lambda i: (0, 0)),
            pl.BlockSpec((Q, D), lambda i: (0, 0)),
            pl.BlockSpec((D, BK), lambda i: (0, i)),
            pl.BlockSpec((1, BK), lambda i: (0, i)),
        ],
        out_specs=pl.BlockSpec((Q, 1), lambda i: (0, 0)),
        out_shape=jax.ShapeDtypeStruct((Q, 1), jnp.int32),
        scratch_shapes=[pltpu.VMEM((Q, 128), jnp.float32),
                        pltpu.VMEM((Q, 128), jnp.int32)],
    )(qsq, qm2, keys_t, ksq)
    return out[:, 0]
